# baseline (speedup 1.0000x reference)
import functools

import jax
import jax.numpy as jnp
import numpy as np
from jax import lax
from jax.experimental import pallas as pl
from jax.experimental.pallas import tpu as pltpu
from jax.experimental.pallas import tpu_sc as plsc

N = 10000
E = 320000
D = 128
DE = 16
NG = 64
NC = 10

NH = N // 2
AR = 5120
GR = NH
EPW = E // 16
CH = 80
NCHUNK = EPW // CH
NPAIR = NCHUNK // 2
NQUAD = (NPAIR - 1) // 2

_BN = 2000
_BE = 8000

def _leaky(v):
    return jnp.where(v >= 0, v, 0.01 * v)


def _edge_agg_body(h_hbm, ea_hbm, src_hbm, dst_hbm, out_hbm,
                   isrc0, isrc1, idst0, idst1, rows0, rows1, msg0, msg1,
                   msgf_v, stage_v, acc_sh, sg0, sg1, se0, se1, si0, si1):
    c = lax.axis_index("c")
    s = lax.axis_index("s")
    base = s * 320
    zero16 = jnp.zeros((16,), jnp.float32)
    cbase = jnp.full((16,), c * NH, jnp.int32)
    grv = jnp.full((16,), GR, jnp.int32)

    def zrow(r, carry):
        for k in range(D // 16):
            stage_v[r, pl.ds(k * 16, 16)] = zero16
        return carry

    lax.fori_loop(0, 16, zrow, 0)

    def zacc(q, carry):
        pltpu.sync_copy(stage_v, acc_sh.at[pl.ds(base + q * 16, 16)])
        return carry

    lax.fori_loop(0, 20, zacc, 0)
    plsc.subcore_barrier()

    ibufs = ((isrc0, idst0, si0), (isrc1, idst1, si1))
    bufs = ((rows0, msg0, sg0, se0), (rows1, msg1, sg1, se1))

    def istart(p, ib):
        isrc, idst, si = ibufs[ib]
        pltpu.async_copy(src_hbm.at[s, p], isrc, si)
        pltpu.async_copy(dst_hbm.at[s, p], idst, si)

    def iwait(p, ib):
        isrc, idst, si = ibufs[ib]
        pltpu.make_async_copy(src_hbm.at[s, p], isrc, si).wait()
        pltpu.make_async_copy(dst_hbm.at[s, p], idst, si).wait()
        for r in range(2):
            for k in range(CH // 16):
                sl = pl.ds(k * 16, 16)
                v = idst[r, sl] - cbase
                ok = (v >= 0) & (v < NH)
                idst[r, sl] = jnp.where(ok, v, grv)

    MJ = CH // 16

    def start(j, b, ib, half):
        rows, msg, sg, se = bufs[b]
        isrc = ibufs[ib][0]
        pltpu.async_copy(h_hbm.at[isrc.at[half]], rows, sg)
        pltpu.async_copy(ea_hbm.at[pl.ds(s * (EPW // 16) + j * MJ, MJ)], msg,
                         se)

    def finish(j, b, ib, half):
        rows, msg, sg, se = bufs[b]
        isrc, idst, si = ibufs[ib]
        pltpu.make_async_copy(h_hbm.at[isrc.at[half]], rows, sg).wait()
        pltpu.make_async_copy(ea_hbm.at[pl.ds(s * (EPW // 16) + j * MJ, MJ)],
                              msg, se).wait()

        def erow(m, mc):
            def irow(i, ic):
                r = m * 16 + i
                for k in range(D // 32):
                    vw = msg[m, i, pl.ds(k * 16, 16)]
                    lo = plsc.bitcast(vw << 16, jnp.float32)
                    hi = plsc.bitcast(vw & jnp.int32(-65536), jnp.float32)
                    sl0 = pl.ds(k * 16, 16)
                    sl1 = pl.ds(64 + k * 16, 16)
                    msgf_v[r, sl0] = jnp.maximum(rows[r, sl0] + lo, 0.0)
                    msgf_v[r, sl1] = jnp.maximum(rows[r, sl1] + hi, 0.0)
                return ic

            lax.fori_loop(0, 16, irow, 0)
            return mc

        lax.fori_loop(0, MJ, erow, 0)
        pltpu.sync_copy(msgf_v, acc_sh.at[idst.at[half]], add=True)

    istart(0, 0)
    iwait(0, 0)
    start(0, 0, 0, 0)

    def quad(t, carry):
        j = 4 * t
        istart(2 * t + 1, 1)
        start(j + 1, 1, 0, 1)
        finish(j, 0, 0, 0)
        iwait(2 * t + 1, 1)
        start(j + 2, 0, 1, 0)
        finish(j + 1, 1, 0, 1)
        istart(2 * t + 2, 0)
        start(j + 3, 1, 1, 1)
        finish(j + 2, 0, 1, 0)
        iwait(2 * t + 2, 0)
        start(j + 4, 0, 0, 0)
        finish(j + 3, 1, 1, 1)
        return carry

    lax.fori_loop(0, NQUAD, quad, 0)

    start(NCHUNK - 1, 1, 0, 1)
    finish(NCHUNK - 2, 0, 0, 0)
    finish(NCHUNK - 1, 1, 0, 1)

    plsc.subcore_barrier()

    ob = c * AR + base

    def outq(q, carry):
        pltpu.sync_copy(acc_sh.at[pl.ds(base + q * 16, 16)], stage_v)
        pltpu.sync_copy(stage_v, out_hbm.at[pl.ds(ob + q * 16, 16)])
        return carry

    lax.fori_loop(0, 20, outq, 0)


@functools.cache
def _edge_agg_kernel():
    mesh = plsc.VectorSubcoreMesh(core_axis_name="c", subcore_axis_name="s",
                                  num_cores=2, num_subcores=16)
    return pl.kernel(
        _edge_agg_body,
        out_type=jax.ShapeDtypeStruct((2 * AR, D), jnp.float32),
        mesh=mesh,
        compiler_params=pltpu.CompilerParams(needs_layout_passes=False),
        scratch_types=[
            pltpu.VMEM((2, CH), jnp.int32),
            pltpu.VMEM((2, CH), jnp.int32),
            pltpu.VMEM((2, CH), jnp.int32),
            pltpu.VMEM((2, CH), jnp.int32),
            pltpu.VMEM((CH, D), jnp.float32),
            pltpu.VMEM((CH, D), jnp.float32),
            pltpu.VMEM((CH // 16, 16, D // 2), jnp.int32),
            pltpu.VMEM((CH // 16, 16, D // 2), jnp.int32),
            pltpu.VMEM((CH, D), jnp.float32),
            pltpu.VMEM((16, D), jnp.float32),
            pltpu.VMEM_SHARED((AR, D), jnp.float32),
            pltpu.SemaphoreType.DMA,
            pltpu.SemaphoreType.DMA,
            pltpu.SemaphoreType.DMA,
            pltpu.SemaphoreType.DMA,
            pltpu.SemaphoreType.DMA,
            pltpu.SemaphoreType.DMA,
        ],
    )


def _edge_agg(h, ea, src4, dst4):
    return _edge_agg_kernel()(h, ea, src4, dst4)


def _mm_bias_body(a_ref, w_ref, b_ref, o_ref):
    o_ref[...] = (jnp.dot(a_ref[...], w_ref[...],
                          preferred_element_type=jnp.float32) + b_ref[...])


def _ea_body(a_ref, w_ref, b_ref, o_ref):
    y = (jnp.dot(a_ref[...], w_ref[...], preferred_element_type=jnp.float32)
         + b_ref[...])
    u = lax.bitcast_convert_type(y, jnp.int32)
    r = (u + 32767 + ((u >> 16) & 1)) >> 16
    w = (r[:, :D // 2] & 65535) | (r[:, D // 2:] << 16)
    o_ref[...] = w.reshape(_BE // 16, 16, D // 2)


def _ea_call(ea, wet, be):
    return pl.pallas_call(
        _ea_body,
        grid=(E // _BE,),
        in_specs=[pl.BlockSpec((_BE, DE), lambda i: (i, 0)),
                  pl.BlockSpec((DE, D), lambda i: (0, 0)),
                  pl.BlockSpec((1, D), lambda i: (0, 0))],
        out_specs=pl.BlockSpec((_BE // 16, 16, D // 2), lambda i: (i, 0, 0)),
        out_shape=jax.ShapeDtypeStruct((E // 16, 16, D // 2), jnp.int32),
    )(ea, wet, be)


def _d1_body(h_ref, agg_ref, eps_ref, w_ref, b_ref,
             y_ref, s_ref, q_ref, accs, accq):
    i = pl.program_id(0)
    z = h_ref[...] * eps_ref[...] + agg_ref[...]
    y = jnp.dot(z, w_ref[...], preferred_element_type=jnp.float32) + b_ref[...]
    y_ref[...] = y

    @pl.when(i == 0)
    def _():
        accs[...] = jnp.zeros_like(accs)
        accq[...] = jnp.zeros_like(accq)

    accs[...] += jnp.sum(y, axis=0, keepdims=True)
    accq[...] += jnp.sum(y * y, axis=0, keepdims=True)
    s_ref[...] = accs[...]
    q_ref[...] = accq[...]


def _d1_call(h, agg, epsv, w1t, b1):
    return pl.pallas_call(
        _d1_body,
        grid=(N // _BN,),
        in_specs=[pl.BlockSpec((_BN, D), lambda i: (i, 0)),
                  pl.BlockSpec((_BN, D), lambda i: (i, 0)),
                  pl.BlockSpec((1, D), lambda i: (0, 0)),
                  pl.BlockSpec((D, D), lambda i: (0, 0)),
                  pl.BlockSpec((1, D), lambda i: (0, 0))],
        out_specs=[pl.BlockSpec((_BN, D), lambda i: (i, 0)),
                   pl.BlockSpec((1, D), lambda i: (0, 0)),
                   pl.BlockSpec((1, D), lambda i: (0, 0))],
        out_shape=[jax.ShapeDtypeStruct((N, D), jnp.float32),
                   jax.ShapeDtypeStruct((1, D), jnp.float32),
                   jax.ShapeDtypeStruct((1, D), jnp.float32)],
        scratch_shapes=[pltpu.VMEM((1, D), jnp.float32),
                        pltpu.VMEM((1, D), jnp.float32)],
    )(h, agg, epsv, w1t, b1)


def _d2_body(y1_ref, a_ref, c_ref, w_ref, b_ref, y_ref, s_ref, q_ref,
             accs, accq):
    i = pl.program_id(0)
    t = _leaky(y1_ref[...] * a_ref[...] + c_ref[...])
    y = jnp.dot(t, w_ref[...], preferred_element_type=jnp.float32) + b_ref[...]
    y_ref[...] = y

    @pl.when(i == 0)
    def _():
        accs[...] = jnp.zeros_like(accs)
        accq[...] = jnp.zeros_like(accq)

    accs[...] += jnp.sum(y, axis=0, keepdims=True)
    accq[...] += jnp.sum(y * y, axis=0, keepdims=True)
    s_ref[...] = accs[...]
    q_ref[...] = accq[...]


def _d2_call(y1, av, cv, w2t, b2):
    return pl.pallas_call(
        _d2_body,
        grid=(N // _BN,),
        in_specs=[pl.BlockSpec((_BN, D), lambda i: (i, 0)),
                  pl.BlockSpec((1, D), lambda i: (0, 0)),
                  pl.BlockSpec((1, D), lambda i: (0, 0)),
                  pl.BlockSpec((D, D), lambda i: (0, 0)),
                  pl.BlockSpec((1, D), lambda i: (0, 0))],
        out_specs=[pl.BlockSpec((_BN, D), lambda i: (i, 0)),
                   pl.BlockSpec((1, D), lambda i: (0, 0)),
                   pl.BlockSpec((1, D), lambda i: (0, 0))],
        out_shape=[jax.ShapeDtypeStruct((N, D), jnp.float32),
                   jax.ShapeDtypeStruct((1, D), jnp.float32),
                   jax.ShapeDtypeStruct((1, D), jnp.float32)],
        scratch_shapes=[pltpu.VMEM((1, D), jnp.float32),
                        pltpu.VMEM((1, D), jnp.float32)],
    )(y1, av, cv, w2t, b2)


def _d3_body(y_ref, a_ref, c_ref, o_ref):
    o_ref[...] = _leaky(y_ref[...] * a_ref[...] + c_ref[...])


def _d3_call(y, av, cv):
    return pl.pallas_call(
        _d3_body,
        grid=(N // _BN,),
        in_specs=[pl.BlockSpec((_BN, D), lambda i: (i, 0)),
                  pl.BlockSpec((1, D), lambda i: (0, 0)),
                  pl.BlockSpec((1, D), lambda i: (0, 0))],
        out_specs=pl.BlockSpec((_BN, D), lambda i: (i, 0)),
        out_shape=jax.ShapeDtypeStruct((N, D), jnp.float32),
    )(y, av, cv)


def _pool_body(bbt_ref, h_ref, pool_ref):
    iot = lax.broadcasted_iota(jnp.int32, (NG, N), 0)
    oht = jnp.where(bbt_ref[...] == iot, 1.0, 0.0)
    pool_ref[...] = jnp.dot(oht, h_ref[...], preferred_element_type=jnp.float32)


def _pool_call(bbt, h):
    return pl.pallas_call(
        _pool_body,
        grid=(1,),
        in_specs=[pl.BlockSpec((NG, N), lambda i: (0, 0)),
                  pl.BlockSpec((N, D), lambda i: (0, 0))],
        out_specs=pl.BlockSpec((NG, D), lambda i: (0, 0)),
        out_shape=jax.ShapeDtypeStruct((NG, D), jnp.float32),
    )(bbt, h)


def _pp_call(pool, wm0rt, bm0):
    return pl.pallas_call(
        _mm_bias_body,
        grid=(1,),
        in_specs=[pl.BlockSpec((NG, D), lambda i: (0, 0)),
                  pl.BlockSpec((D, D), lambda i: (0, 0)),
                  pl.BlockSpec((1, D), lambda i: (0, 0))],
        out_specs=pl.BlockSpec((NG, D), lambda i: (0, 0)),
        out_shape=jax.ShapeDtypeStruct((NG, D), jnp.float32),
    )(pool, wm0rt, bm0)


def _head_body(h_ref, bb_ref, pp_ref, w0l_ref, w1_ref, wf_ref,
               bm1_ref, bf_ref, o_ref):
    iot = lax.broadcasted_iota(jnp.int32, (_BN, NG), 1)
    oh = jnp.where(bb_ref[...] == iot, 1.0, 0.0)
    r = (jnp.dot(h_ref[...], w0l_ref[...], preferred_element_type=jnp.float32)
         + jnp.dot(oh, pp_ref[...], preferred_element_type=jnp.float32))
    t = _leaky(r)
    u = _leaky(jnp.dot(t, w1_ref[...], preferred_element_type=jnp.float32)
               + bm1_ref[...])
    o_ref[...] = (jnp.dot(u, wf_ref[...], preferred_element_type=jnp.float32)
                  + bf_ref[...])


def _head_call(h, bb, pp, w0lt, wm1t, wft, bm1, bf):
    return pl.pallas_call(
        _head_body,
        grid=(N // _BN,),
        in_specs=[pl.BlockSpec((_BN, D), lambda i: (i, 0)),
                  pl.BlockSpec((_BN, NG), lambda i: (i, 0)),
                  pl.BlockSpec((NG, D), lambda i: (0, 0)),
                  pl.BlockSpec((D, D), lambda i: (0, 0)),
                  pl.BlockSpec((D, D), lambda i: (0, 0)),
                  pl.BlockSpec((D, D), lambda i: (0, 0)),
                  pl.BlockSpec((1, D), lambda i: (0, 0)),
                  pl.BlockSpec((1, D), lambda i: (0, 0))],
        out_specs=pl.BlockSpec((_BN, D), lambda i: (i, 0)),
        out_shape=jax.ShapeDtypeStruct((N, D), jnp.float32),
    )(h, bb, pp, w0lt, wm1t, wft, bm1, bf)


def _bn_consts(s, q, g, b):
    m = s / N
    v = q / N - m * m
    inv = g[None, :] / jnp.sqrt(v + 1e-5)
    return inv, b[None, :] - m * inv


def kernel(x, edge_index, edge_attr, batch, params):
    src4 = edge_index[0].reshape(16, NPAIR, 2, CH)
    dst4 = edge_index[1].reshape(16, NPAIR, 2, CH)
    bb = jnp.broadcast_to(batch[:, None], (N, NG))
    bbt = jnp.broadcast_to(batch[None, :], (NG, N))

    h = x
    for l in range(2):
        p = params['conv%d' % l]
        ea = _ea_call(edge_attr, p['We'].T, p['be'][None])
        parts = _edge_agg(h, ea, src4, dst4)
        agg = jnp.concatenate([parts[:NH], parts[AR:AR + NH]], axis=0)
        epsv = jnp.full((1, D), 1.0 + p['eps'], jnp.float32)
        y1, s1, q1 = _d1_call(h, agg, epsv, p['W1'].T, p['b1'][None])
        av1, cv1 = _bn_consts(s1, q1, p['gamma1'], p['beta1'])
        y2, s2, q2 = _d2_call(y1, av1, cv1, p['W2'].T, p['b2'][None])
        av2, cv2 = _bn_consts(s2, q2, params['bn%d_g' % l],
                              params['bn%d_b' % l])
        h = _d3_call(y2, av2, cv2)

    pool = _pool_call(bbt, h)
    pp = _pp_call(pool, params['Wm0'][:, D:].T, params['bm0'][None])
    wft = jnp.zeros((D, D), jnp.float32).at[:, :NC].set(params['Wf'].T)
    bf = jnp.zeros((1, D), jnp.float32).at[0, :NC].set(params['bf'])
    out = _head_call(h, bb, pp, params['Wm0'][:, :D].T, params['Wm1'].T,
                     wft, params['bm1'][None], bf)
    return out[:, :NC]


# speedup vs baseline: 2.0207x; 2.0207x over previous
import functools

import jax
import jax.numpy as jnp
import numpy as np
from jax import lax
from jax.experimental import pallas as pl
from jax.experimental.pallas import tpu as pltpu
from jax.experimental.pallas import tpu_sc as plsc

N = 10000
E = 320000
D = 128
DE = 16
NG = 64
NC = 10

NH = N // 2
AR = 5120
GR = NH
EPW = E // 16
CH = 80
NCHUNK = EPW // CH
NPAIR = NCHUNK // 2
NQUAD = (NPAIR - 1) // 2

_BN = 2000
_BE = 8000

def _leaky(v):
    return jnp.where(v >= 0, v, 0.01 * v)


def _edge_agg_body(h_hbm, ea_hbm, src_hbm, dst_hbm, out_hbm,
                   isrc0, isrc1, idst0, idst1, rows0, rows1, msg0, msg1,
                   stage_v, acc_sh, sg0, sg1, se0, se1, si0, si1):
    c = lax.axis_index("c")
    s = lax.axis_index("s")
    base = s * 320
    zero16 = jnp.zeros((16,), jnp.float32)
    cbase = jnp.full((16,), c * NH, jnp.int32)
    grv = jnp.full((16,), GR, jnp.int32)

    def zrow(r, carry):
        for k in range(D // 16):
            stage_v[r, pl.ds(k * 16, 16)] = zero16
        return carry

    lax.fori_loop(0, 16, zrow, 0)

    def zacc(q, carry):
        pltpu.sync_copy(stage_v, acc_sh.at[pl.ds(base + q * 16, 16)])
        return carry

    lax.fori_loop(0, 20, zacc, 0)
    plsc.subcore_barrier()

    ibufs = ((isrc0, idst0, si0), (isrc1, idst1, si1))
    bufs = ((rows0, msg0, sg0, se0), (rows1, msg1, sg1, se1))

    def istart(p, ib):
        isrc, idst, si = ibufs[ib]
        pltpu.async_copy(src_hbm.at[s, p], isrc, si)
        pltpu.async_copy(dst_hbm.at[s, p], idst, si)

    def iwait(p, ib):
        isrc, idst, si = ibufs[ib]
        pltpu.make_async_copy(src_hbm.at[s, p], isrc, si).wait()
        pltpu.make_async_copy(dst_hbm.at[s, p], idst, si).wait()
        for r in range(2):
            for k in range(CH // 16):
                sl = pl.ds(k * 16, 16)
                v = idst[r, sl] - cbase
                ok = (v >= 0) & (v < NH)
                idst[r, sl] = jnp.where(ok, v, grv)

    def start(j, b, ib, half):
        rows, msg, sg, se = bufs[b]
        isrc = ibufs[ib][0]
        pltpu.async_copy(h_hbm.at[isrc.at[half]], rows, sg)
        pltpu.async_copy(ea_hbm.at[pl.ds(s * EPW + j * CH, CH)], msg, se)

    def finish(j, b, ib, half):
        rows, msg, sg, se = bufs[b]
        isrc, idst, si = ibufs[ib]
        pltpu.make_async_copy(h_hbm.at[isrc.at[half]], rows, sg).wait()
        pltpu.make_async_copy(ea_hbm.at[pl.ds(s * EPW + j * CH, CH)], msg,
                              se).wait()

        def row(r, rc):
            for k in range(D // 16):
                sl = pl.ds(k * 16, 16)
                msg[r, sl] = jnp.maximum(rows[r, sl] + msg[r, sl], 0.0)
            return rc

        lax.fori_loop(0, CH, row, 0)
        pltpu.sync_copy(msg, acc_sh.at[idst.at[half]], add=True)

    istart(0, 0)
    iwait(0, 0)
    start(0, 0, 0, 0)

    def quad(t, carry):
        j = 4 * t
        istart(2 * t + 1, 1)
        start(j + 1, 1, 0, 1)
        finish(j, 0, 0, 0)
        iwait(2 * t + 1, 1)
        start(j + 2, 0, 1, 0)
        finish(j + 1, 1, 0, 1)
        istart(2 * t + 2, 0)
        start(j + 3, 1, 1, 1)
        finish(j + 2, 0, 1, 0)
        iwait(2 * t + 2, 0)
        start(j + 4, 0, 0, 0)
        finish(j + 3, 1, 1, 1)
        return carry

    lax.fori_loop(0, NQUAD, quad, 0)

    start(NCHUNK - 1, 1, 0, 1)
    finish(NCHUNK - 2, 0, 0, 0)
    finish(NCHUNK - 1, 1, 0, 1)

    plsc.subcore_barrier()

    ob = c * AR + base

    def outq(q, carry):
        pltpu.sync_copy(acc_sh.at[pl.ds(base + q * 16, 16)], stage_v)
        pltpu.sync_copy(stage_v, out_hbm.at[pl.ds(ob + q * 16, 16)])
        return carry

    lax.fori_loop(0, 20, outq, 0)


@functools.cache
def _edge_agg_kernel():
    mesh = plsc.VectorSubcoreMesh(core_axis_name="c", subcore_axis_name="s",
                                  num_cores=2, num_subcores=16)
    return pl.kernel(
        _edge_agg_body,
        out_type=jax.ShapeDtypeStruct((2 * AR, D), jnp.float32),
        mesh=mesh,
        scratch_types=[
            pltpu.VMEM((2, CH), jnp.int32),
            pltpu.VMEM((2, CH), jnp.int32),
            pltpu.VMEM((2, CH), jnp.int32),
            pltpu.VMEM((2, CH), jnp.int32),
            pltpu.VMEM((CH, D), jnp.float32),
            pltpu.VMEM((CH, D), jnp.float32),
            pltpu.VMEM((CH, D), jnp.float32),
            pltpu.VMEM((CH, D), jnp.float32),
            pltpu.VMEM((16, D), jnp.float32),
            pltpu.VMEM_SHARED((AR, D), jnp.float32),
            pltpu.SemaphoreType.DMA,
            pltpu.SemaphoreType.DMA,
            pltpu.SemaphoreType.DMA,
            pltpu.SemaphoreType.DMA,
            pltpu.SemaphoreType.DMA,
            pltpu.SemaphoreType.DMA,
        ],
    )


def _edge_agg(h, ea, src4, dst4):
    return _edge_agg_kernel()(h, ea, src4, dst4)


def _mm_bias_body(a_ref, w_ref, b_ref, o_ref):
    o_ref[...] = (jnp.dot(a_ref[...], w_ref[...],
                          preferred_element_type=jnp.float32, precision=lax.Precision.HIGHEST) + b_ref[...])


def _ea_call(ea, wet, be):
    return pl.pallas_call(
        _mm_bias_body,
        grid=(E // _BE,),
        in_specs=[pl.BlockSpec((_BE, DE), lambda i: (i, 0)),
                  pl.BlockSpec((DE, D), lambda i: (0, 0)),
                  pl.BlockSpec((1, D), lambda i: (0, 0))],
        out_specs=pl.BlockSpec((_BE, D), lambda i: (i, 0)),
        out_shape=jax.ShapeDtypeStruct((E, D), jnp.float32),
    )(ea, wet, be)


def _d1_body(h_ref, agg_ref, eps_ref, w_ref, b_ref,
             y_ref, s_ref, q_ref, accs, accq):
    i = pl.program_id(0)
    z = h_ref[...] * eps_ref[...] + agg_ref[...]
    y = jnp.dot(z, w_ref[...], preferred_element_type=jnp.float32, precision=lax.Precision.HIGHEST) + b_ref[...]
    y_ref[...] = y

    @pl.when(i == 0)
    def _():
        accs[...] = jnp.zeros_like(accs)
        accq[...] = jnp.zeros_like(accq)

    accs[...] += jnp.sum(y, axis=0, keepdims=True)
    accq[...] += jnp.sum(y * y, axis=0, keepdims=True)
    s_ref[...] = accs[...]
    q_ref[...] = accq[...]


def _d1_call(h, agg, epsv, w1t, b1):
    return pl.pallas_call(
        _d1_body,
        grid=(N // _BN,),
        in_specs=[pl.BlockSpec((_BN, D), lambda i: (i, 0)),
                  pl.BlockSpec((_BN, D), lambda i: (i, 0)),
                  pl.BlockSpec((1, D), lambda i: (0, 0)),
                  pl.BlockSpec((D, D), lambda i: (0, 0)),
                  pl.BlockSpec((1, D), lambda i: (0, 0))],
        out_specs=[pl.BlockSpec((_BN, D), lambda i: (i, 0)),
                   pl.BlockSpec((1, D), lambda i: (0, 0)),
                   pl.BlockSpec((1, D), lambda i: (0, 0))],
        out_shape=[jax.ShapeDtypeStruct((N, D), jnp.float32),
                   jax.ShapeDtypeStruct((1, D), jnp.float32),
                   jax.ShapeDtypeStruct((1, D), jnp.float32)],
        scratch_shapes=[pltpu.VMEM((1, D), jnp.float32),
                        pltpu.VMEM((1, D), jnp.float32)],
    )(h, agg, epsv, w1t, b1)


def _d2_body(y1_ref, a_ref, c_ref, w_ref, b_ref, y_ref, s_ref, q_ref,
             accs, accq):
    i = pl.program_id(0)
    t = _leaky(y1_ref[...] * a_ref[...] + c_ref[...])
    y = jnp.dot(t, w_ref[...], preferred_element_type=jnp.float32, precision=lax.Precision.HIGHEST) + b_ref[...]
    y_ref[...] = y

    @pl.when(i == 0)
    def _():
        accs[...] = jnp.zeros_like(accs)
        accq[...] = jnp.zeros_like(accq)

    accs[...] += jnp.sum(y, axis=0, keepdims=True)
    accq[...] += jnp.sum(y * y, axis=0, keepdims=True)
    s_ref[...] = accs[...]
    q_ref[...] = accq[...]


def _d2_call(y1, av, cv, w2t, b2):
    return pl.pallas_call(
        _d2_body,
        grid=(N // _BN,),
        in_specs=[pl.BlockSpec((_BN, D), lambda i: (i, 0)),
                  pl.BlockSpec((1, D), lambda i: (0, 0)),
                  pl.BlockSpec((1, D), lambda i: (0, 0)),
                  pl.BlockSpec((D, D), lambda i: (0, 0)),
                  pl.BlockSpec((1, D), lambda i: (0, 0))],
        out_specs=[pl.BlockSpec((_BN, D), lambda i: (i, 0)),
                   pl.BlockSpec((1, D), lambda i: (0, 0)),
                   pl.BlockSpec((1, D), lambda i: (0, 0))],
        out_shape=[jax.ShapeDtypeStruct((N, D), jnp.float32),
                   jax.ShapeDtypeStruct((1, D), jnp.float32),
                   jax.ShapeDtypeStruct((1, D), jnp.float32)],
        scratch_shapes=[pltpu.VMEM((1, D), jnp.float32),
                        pltpu.VMEM((1, D), jnp.float32)],
    )(y1, av, cv, w2t, b2)


def _d3_body(y_ref, a_ref, c_ref, o_ref):
    o_ref[...] = _leaky(y_ref[...] * a_ref[...] + c_ref[...])


def _d3_call(y, av, cv):
    return pl.pallas_call(
        _d3_body,
        grid=(N // _BN,),
        in_specs=[pl.BlockSpec((_BN, D), lambda i: (i, 0)),
                  pl.BlockSpec((1, D), lambda i: (0, 0)),
                  pl.BlockSpec((1, D), lambda i: (0, 0))],
        out_specs=pl.BlockSpec((_BN, D), lambda i: (i, 0)),
        out_shape=jax.ShapeDtypeStruct((N, D), jnp.float32),
    )(y, av, cv)


def _pool_body(bbt_ref, h_ref, pool_ref):
    iot = lax.broadcasted_iota(jnp.int32, (NG, N), 0)
    oht = jnp.where(bbt_ref[...] == iot, 1.0, 0.0)
    pool_ref[...] = jnp.dot(oht, h_ref[...], preferred_element_type=jnp.float32, precision=lax.Precision.HIGHEST)


def _pool_call(bbt, h):
    return pl.pallas_call(
        _pool_body,
        grid=(1,),
        in_specs=[pl.BlockSpec((NG, N), lambda i: (0, 0)),
                  pl.BlockSpec((N, D), lambda i: (0, 0))],
        out_specs=pl.BlockSpec((NG, D), lambda i: (0, 0)),
        out_shape=jax.ShapeDtypeStruct((NG, D), jnp.float32),
    )(bbt, h)


def _pp_call(pool, wm0rt, bm0):
    return pl.pallas_call(
        _mm_bias_body,
        grid=(1,),
        in_specs=[pl.BlockSpec((NG, D), lambda i: (0, 0)),
                  pl.BlockSpec((D, D), lambda i: (0, 0)),
                  pl.BlockSpec((1, D), lambda i: (0, 0))],
        out_specs=pl.BlockSpec((NG, D), lambda i: (0, 0)),
        out_shape=jax.ShapeDtypeStruct((NG, D), jnp.float32),
    )(pool, wm0rt, bm0)


def _head_body(h_ref, bb_ref, pp_ref, w0l_ref, w1_ref, wf_ref,
               bm1_ref, bf_ref, o_ref):
    iot = lax.broadcasted_iota(jnp.int32, (_BN, NG), 1)
    oh = jnp.where(bb_ref[...] == iot, 1.0, 0.0)
    r = (jnp.dot(h_ref[...], w0l_ref[...], preferred_element_type=jnp.float32, precision=lax.Precision.HIGHEST)
         + jnp.dot(oh, pp_ref[...], preferred_element_type=jnp.float32, precision=lax.Precision.HIGHEST))
    t = _leaky(r)
    u = _leaky(jnp.dot(t, w1_ref[...], preferred_element_type=jnp.float32, precision=lax.Precision.HIGHEST)
               + bm1_ref[...])
    o_ref[...] = (jnp.dot(u, wf_ref[...], preferred_element_type=jnp.float32, precision=lax.Precision.HIGHEST)
                  + bf_ref[...])


def _head_call(h, bb, pp, w0lt, wm1t, wft, bm1, bf):
    return pl.pallas_call(
        _head_body,
        grid=(N // _BN,),
        in_specs=[pl.BlockSpec((_BN, D), lambda i: (i, 0)),
                  pl.BlockSpec((_BN, NG), lambda i: (i, 0)),
                  pl.BlockSpec((NG, D), lambda i: (0, 0)),
                  pl.BlockSpec((D, D), lambda i: (0, 0)),
                  pl.BlockSpec((D, D), lambda i: (0, 0)),
                  pl.BlockSpec((D, D), lambda i: (0, 0)),
                  pl.BlockSpec((1, D), lambda i: (0, 0)),
                  pl.BlockSpec((1, D), lambda i: (0, 0))],
        out_specs=pl.BlockSpec((_BN, D), lambda i: (i, 0)),
        out_shape=jax.ShapeDtypeStruct((N, D), jnp.float32),
    )(h, bb, pp, w0lt, wm1t, wft, bm1, bf)


def _bn_consts(s, q, g, b):
    m = s / N
    v = q / N - m * m
    inv = g[None, :] / jnp.sqrt(v + 1e-5)
    return inv, b[None, :] - m * inv


def kernel(x, edge_index, edge_attr, batch, params):
    src4 = edge_index[0].reshape(16, NPAIR, 2, CH)
    dst4 = edge_index[1].reshape(16, NPAIR, 2, CH)
    bb = jnp.broadcast_to(batch[:, None], (N, NG))
    bbt = jnp.broadcast_to(batch[None, :], (NG, N))

    h = x
    for l in range(2):
        p = params['conv%d' % l]
        ea = _ea_call(edge_attr, p['We'].T, p['be'][None])
        parts = _edge_agg(h, ea, src4, dst4)
        agg = jnp.concatenate([parts[:NH], parts[AR:AR + NH]], axis=0)
        epsv = jnp.full((1, D), 1.0 + p['eps'], jnp.float32)
        y1, s1, q1 = _d1_call(h, agg, epsv, p['W1'].T, p['b1'][None])
        av1, cv1 = _bn_consts(s1, q1, p['gamma1'], p['beta1'])
        y2, s2, q2 = _d2_call(y1, av1, cv1, p['W2'].T, p['b2'][None])
        av2, cv2 = _bn_consts(s2, q2, params['bn%d_g' % l],
                              params['bn%d_b' % l])
        h = _d3_call(y2, av2, cv2)

    pool = _pool_call(bbt, h)
    pp = _pp_call(pool, params['Wm0'][:, D:].T, params['bm0'][None])
    wft = jnp.zeros((D, D), jnp.float32).at[:, :NC].set(params['Wf'].T)
    bf = jnp.zeros((1, D), jnp.float32).at[0, :NC].set(params['bf'])
    out = _head_call(h, bb, pp, params['Wm0'][:, :D].T, params['Wm1'].T,
                     wft, params['bm1'][None], bf)
    return out[:, :NC]


# speedup vs baseline: 2.9707x; 1.4701x over previous
import functools

import jax
import jax.numpy as jnp
import numpy as np
from jax import lax
from jax.experimental import pallas as pl
from jax.experimental.pallas import tpu as pltpu
from jax.experimental.pallas import tpu_sc as plsc

N = 10000
E = 320000
D = 128
DE = 16
NG = 64
NC = 10

AR = 10240
RPT = AR // 16
EPW = E // 32
CH = 40
NCHUNK = EPW // CH
NPAIR = NCHUNK // 2
NQUAD = (NPAIR - 1) // 2

_BN = 2000
_BE = 8000

def _leaky(v):
    return jnp.where(v >= 0, v, 0.01 * v)


def _edge_agg_body(h_hbm, ea_hbm, src_hbm, dst_hbm, out_hbm,
                   isrc0, isrc1, idst0, idst1, rows0, rows1, msg0, msg1,
                   stage_v, acc_sh, sg0, sg1, se0, se1, si0, si1):
    c = lax.axis_index("c")
    s = lax.axis_index("s")
    w = c * 16 + s
    base = s * RPT
    zero16 = jnp.zeros((16,), jnp.float32)

    def zrow(r, carry):
        for k in range(D // 16):
            stage_v[r, pl.ds(k * 16, 16)] = zero16
        return carry

    lax.fori_loop(0, 16, zrow, 0)

    def zacc(q, carry):
        pltpu.sync_copy(stage_v, acc_sh.at[pl.ds(base + q * 16, 16)])
        return carry

    lax.fori_loop(0, RPT // 16, zacc, 0)
    plsc.subcore_barrier()

    ibufs = ((isrc0, idst0, si0), (isrc1, idst1, si1))
    bufs = ((rows0, msg0, sg0, se0), (rows1, msg1, sg1, se1))

    def istart(p, ib):
        isrc, idst, si = ibufs[ib]
        pltpu.async_copy(src_hbm.at[w, p], isrc, si)
        pltpu.async_copy(dst_hbm.at[w, p], idst, si)

    def iwait(p, ib):
        isrc, idst, si = ibufs[ib]
        pltpu.make_async_copy(src_hbm.at[w, p], isrc, si).wait()
        pltpu.make_async_copy(dst_hbm.at[w, p], idst, si).wait()

    def start(j, b, ib, half):
        rows, msg, sg, se = bufs[b]
        isrc = ibufs[ib][0]
        pltpu.async_copy(h_hbm.at[isrc.at[half]], rows, sg)
        pltpu.async_copy(ea_hbm.at[pl.ds(w * EPW + j * CH, CH)], msg, se)

    def finish(j, b, ib, half):
        rows, msg, sg, se = bufs[b]
        isrc, idst, si = ibufs[ib]
        pltpu.make_async_copy(h_hbm.at[isrc.at[half]], rows, sg).wait()
        pltpu.make_async_copy(ea_hbm.at[pl.ds(w * EPW + j * CH, CH)], msg,
                              se).wait()

        def row(r, rc):
            for k in range(D // 16):
                sl = pl.ds(k * 16, 16)
                msg[r, sl] = jnp.maximum(rows[r, sl] + msg[r, sl], 0.0)
            return rc

        lax.fori_loop(0, CH, row, 0)
        pltpu.sync_copy(msg, acc_sh.at[idst.at[half]], add=True)

    istart(0, 0)
    iwait(0, 0)
    start(0, 0, 0, 0)

    def quad(t, carry):
        j = 4 * t
        istart(2 * t + 1, 1)
        start(j + 1, 1, 0, 1)
        finish(j, 0, 0, 0)
        iwait(2 * t + 1, 1)
        start(j + 2, 0, 1, 0)
        finish(j + 1, 1, 0, 1)
        istart(2 * t + 2, 0)
        start(j + 3, 1, 1, 1)
        finish(j + 2, 0, 1, 0)
        iwait(2 * t + 2, 0)
        start(j + 4, 0, 0, 0)
        finish(j + 3, 1, 1, 1)
        return carry

    lax.fori_loop(0, NQUAD, quad, 0)

    start(NCHUNK - 1, 1, 0, 1)
    finish(NCHUNK - 2, 0, 0, 0)
    finish(NCHUNK - 1, 1, 0, 1)

    plsc.subcore_barrier()

    ob = c * AR + base

    def outq(q, carry):
        pltpu.sync_copy(acc_sh.at[pl.ds(base + q * 16, 16)], stage_v)
        pltpu.sync_copy(stage_v, out_hbm.at[pl.ds(ob + q * 16, 16)])
        return carry

    lax.fori_loop(0, RPT // 16, outq, 0)


@functools.cache
def _edge_agg_kernel():
    mesh = plsc.VectorSubcoreMesh(core_axis_name="c", subcore_axis_name="s",
                                  num_cores=2, num_subcores=16)
    return pl.kernel(
        _edge_agg_body,
        out_type=jax.ShapeDtypeStruct((2 * AR, D), jnp.float32),
        mesh=mesh,
        scratch_types=[
            pltpu.VMEM((2, CH), jnp.int32),
            pltpu.VMEM((2, CH), jnp.int32),
            pltpu.VMEM((2, CH), jnp.int32),
            pltpu.VMEM((2, CH), jnp.int32),
            pltpu.VMEM((CH, D), jnp.float32),
            pltpu.VMEM((CH, D), jnp.float32),
            pltpu.VMEM((CH, D), jnp.float32),
            pltpu.VMEM((CH, D), jnp.float32),
            pltpu.VMEM((16, D), jnp.float32),
            pltpu.VMEM_SHARED((AR, D), jnp.float32),
            pltpu.SemaphoreType.DMA,
            pltpu.SemaphoreType.DMA,
            pltpu.SemaphoreType.DMA,
            pltpu.SemaphoreType.DMA,
            pltpu.SemaphoreType.DMA,
            pltpu.SemaphoreType.DMA,
        ],
    )


def _edge_agg(h, ea, src4, dst4):
    return _edge_agg_kernel()(h, ea, src4, dst4)


def _mm_bias_body(a_ref, w_ref, b_ref, o_ref):
    o_ref[...] = (jnp.dot(a_ref[...], w_ref[...],
                          preferred_element_type=jnp.float32) + b_ref[...])


def _ea_call(ea, wet, be):
    return pl.pallas_call(
        _mm_bias_body,
        grid=(E // _BE,),
        in_specs=[pl.BlockSpec((_BE, DE), lambda i: (i, 0)),
                  pl.BlockSpec((DE, D), lambda i: (0, 0)),
                  pl.BlockSpec((1, D), lambda i: (0, 0))],
        out_specs=pl.BlockSpec((_BE, D), lambda i: (i, 0)),
        out_shape=jax.ShapeDtypeStruct((E, D), jnp.float32),
    )(ea, wet, be)


def _d1_body(h_ref, a0_ref, a1_ref, eps_ref, w_ref, b_ref,
             y_ref, s_ref, q_ref, accs, accq):
    i = pl.program_id(0)
    z = h_ref[...] * eps_ref[...] + a0_ref[...] + a1_ref[...]
    y = jnp.dot(z, w_ref[...], preferred_element_type=jnp.float32) + b_ref[...]
    y_ref[...] = y

    @pl.when(i == 0)
    def _():
        accs[...] = jnp.zeros_like(accs)
        accq[...] = jnp.zeros_like(accq)

    accs[...] += jnp.sum(y, axis=0, keepdims=True)
    accq[...] += jnp.sum(y * y, axis=0, keepdims=True)
    s_ref[...] = accs[...]
    q_ref[...] = accq[...]


def _d1_call(h, a0, a1, epsv, w1t, b1):
    return pl.pallas_call(
        _d1_body,
        grid=(N // _BN,),
        in_specs=[pl.BlockSpec((_BN, D), lambda i: (i, 0)),
                  pl.BlockSpec((_BN, D), lambda i: (i, 0)),
                  pl.BlockSpec((_BN, D), lambda i: (i, 0)),
                  pl.BlockSpec((1, D), lambda i: (0, 0)),
                  pl.BlockSpec((D, D), lambda i: (0, 0)),
                  pl.BlockSpec((1, D), lambda i: (0, 0))],
        out_specs=[pl.BlockSpec((_BN, D), lambda i: (i, 0)),
                   pl.BlockSpec((1, D), lambda i: (0, 0)),
                   pl.BlockSpec((1, D), lambda i: (0, 0))],
        out_shape=[jax.ShapeDtypeStruct((N, D), jnp.float32),
                   jax.ShapeDtypeStruct((1, D), jnp.float32),
                   jax.ShapeDtypeStruct((1, D), jnp.float32)],
        scratch_shapes=[pltpu.VMEM((1, D), jnp.float32),
                        pltpu.VMEM((1, D), jnp.float32)],
    )(h, a0, a1, epsv, w1t, b1)


def _d2_body(y1_ref, a_ref, c_ref, w_ref, b_ref, y_ref, s_ref, q_ref,
             accs, accq):
    i = pl.program_id(0)
    t = _leaky(y1_ref[...] * a_ref[...] + c_ref[...])
    y = jnp.dot(t, w_ref[...], preferred_element_type=jnp.float32) + b_ref[...]
    y_ref[...] = y

    @pl.when(i == 0)
    def _():
        accs[...] = jnp.zeros_like(accs)
        accq[...] = jnp.zeros_like(accq)

    accs[...] += jnp.sum(y, axis=0, keepdims=True)
    accq[...] += jnp.sum(y * y, axis=0, keepdims=True)
    s_ref[...] = accs[...]
    q_ref[...] = accq[...]


def _d2_call(y1, av, cv, w2t, b2):
    return pl.pallas_call(
        _d2_body,
        grid=(N // _BN,),
        in_specs=[pl.BlockSpec((_BN, D), lambda i: (i, 0)),
                  pl.BlockSpec((1, D), lambda i: (0, 0)),
                  pl.BlockSpec((1, D), lambda i: (0, 0)),
                  pl.BlockSpec((D, D), lambda i: (0, 0)),
                  pl.BlockSpec((1, D), lambda i: (0, 0))],
        out_specs=[pl.BlockSpec((_BN, D), lambda i: (i, 0)),
                   pl.BlockSpec((1, D), lambda i: (0, 0)),
                   pl.BlockSpec((1, D), lambda i: (0, 0))],
        out_shape=[jax.ShapeDtypeStruct((N, D), jnp.float32),
                   jax.ShapeDtypeStruct((1, D), jnp.float32),
                   jax.ShapeDtypeStruct((1, D), jnp.float32)],
        scratch_shapes=[pltpu.VMEM((1, D), jnp.float32),
                        pltpu.VMEM((1, D), jnp.float32)],
    )(y1, av, cv, w2t, b2)


def _d3_body(y_ref, a_ref, c_ref, o_ref):
    o_ref[...] = _leaky(y_ref[...] * a_ref[...] + c_ref[...])


def _d3_call(y, av, cv):
    return pl.pallas_call(
        _d3_body,
        grid=(N // _BN,),
        in_specs=[pl.BlockSpec((_BN, D), lambda i: (i, 0)),
                  pl.BlockSpec((1, D), lambda i: (0, 0)),
                  pl.BlockSpec((1, D), lambda i: (0, 0))],
        out_specs=pl.BlockSpec((_BN, D), lambda i: (i, 0)),
        out_shape=jax.ShapeDtypeStruct((N, D), jnp.float32),
    )(y, av, cv)


def _pool_body(bbt_ref, h_ref, pool_ref):
    iot = lax.broadcasted_iota(jnp.int32, (NG, N), 0)
    oht = jnp.where(bbt_ref[...] == iot, 1.0, 0.0)
    pool_ref[...] = jnp.dot(oht, h_ref[...], preferred_element_type=jnp.float32)


def _pool_call(bbt, h):
    return pl.pallas_call(
        _pool_body,
        grid=(1,),
        in_specs=[pl.BlockSpec((NG, N), lambda i: (0, 0)),
                  pl.BlockSpec((N, D), lambda i: (0, 0))],
        out_specs=pl.BlockSpec((NG, D), lambda i: (0, 0)),
        out_shape=jax.ShapeDtypeStruct((NG, D), jnp.float32),
    )(bbt, h)


def _pp_call(pool, wm0rt, bm0):
    return pl.pallas_call(
        _mm_bias_body,
        grid=(1,),
        in_specs=[pl.BlockSpec((NG, D), lambda i: (0, 0)),
                  pl.BlockSpec((D, D), lambda i: (0, 0)),
                  pl.BlockSpec((1, D), lambda i: (0, 0))],
        out_specs=pl.BlockSpec((NG, D), lambda i: (0, 0)),
        out_shape=jax.ShapeDtypeStruct((NG, D), jnp.float32),
    )(pool, wm0rt, bm0)


def _head_body(h_ref, bb_ref, pp_ref, w0l_ref, w1_ref, wf_ref,
               bm1_ref, bf_ref, o_ref):
    iot = lax.broadcasted_iota(jnp.int32, (_BN, NG), 1)
    oh = jnp.where(bb_ref[...] == iot, 1.0, 0.0)
    r = (jnp.dot(h_ref[...], w0l_ref[...], preferred_element_type=jnp.float32)
         + jnp.dot(oh, pp_ref[...], preferred_element_type=jnp.float32))
    t = _leaky(r)
    u = _leaky(jnp.dot(t, w1_ref[...], preferred_element_type=jnp.float32)
               + bm1_ref[...])
    o_ref[...] = (jnp.dot(u, wf_ref[...], preferred_element_type=jnp.float32)
                  + bf_ref[...])


def _head_call(h, bb, pp, w0lt, wm1t, wft, bm1, bf):
    return pl.pallas_call(
        _head_body,
        grid=(N // _BN,),
        in_specs=[pl.BlockSpec((_BN, D), lambda i: (i, 0)),
                  pl.BlockSpec((_BN, NG), lambda i: (i, 0)),
                  pl.BlockSpec((NG, D), lambda i: (0, 0)),
                  pl.BlockSpec((D, D), lambda i: (0, 0)),
                  pl.BlockSpec((D, D), lambda i: (0, 0)),
                  pl.BlockSpec((D, D), lambda i: (0, 0)),
                  pl.BlockSpec((1, D), lambda i: (0, 0)),
                  pl.BlockSpec((1, D), lambda i: (0, 0))],
        out_specs=pl.BlockSpec((_BN, D), lambda i: (i, 0)),
        out_shape=jax.ShapeDtypeStruct((N, D), jnp.float32),
    )(h, bb, pp, w0lt, wm1t, wft, bm1, bf)


def _bn_consts(s, q, g, b):
    m = s / N
    v = q / N - m * m
    inv = g[None, :] / jnp.sqrt(v + 1e-5)
    return inv, b[None, :] - m * inv


def kernel(x, edge_index, edge_attr, batch, params):
    src4 = edge_index[0].reshape(32, NPAIR, 2, CH)
    dst4 = edge_index[1].reshape(32, NPAIR, 2, CH)
    bb = jnp.broadcast_to(batch[:, None], (N, NG))
    bbt = jnp.broadcast_to(batch[None, :], (NG, N))

    h = x
    for l in range(2):
        p = params['conv%d' % l]
        ea = _ea_call(edge_attr, p['We'].T, p['be'][None])
        parts = _edge_agg(h, ea, src4, dst4)
        epsv = jnp.full((1, D), 1.0 + p['eps'], jnp.float32)
        y1, s1, q1 = _d1_call(h, parts[:N], parts[AR:AR + N], epsv,
                              p['W1'].T, p['b1'][None])
        av1, cv1 = _bn_consts(s1, q1, p['gamma1'], p['beta1'])
        y2, s2, q2 = _d2_call(y1, av1, cv1, p['W2'].T, p['b2'][None])
        av2, cv2 = _bn_consts(s2, q2, params['bn%d_g' % l],
                              params['bn%d_b' % l])
        h = _d3_call(y2, av2, cv2)

    pool = _pool_call(bbt, h)
    pp = _pp_call(pool, params['Wm0'][:, D:].T, params['bm0'][None])
    wft = jnp.zeros((D, D), jnp.float32).at[:, :NC].set(params['Wf'].T)
    bf = jnp.zeros((1, D), jnp.float32).at[0, :NC].set(params['bf'])
    out = _head_call(h, bb, pp, params['Wm0'][:, :D].T, params['Wm1'].T,
                     wft, params['bm1'][None], bf)
    return out[:, :NC]
